# baseline (speedup 1.0000x reference)
import functools

import numpy as np
import jax
import jax.numpy as jnp
from jax import lax
from jax.experimental import pallas as pl
from jax.experimental.pallas import tpu as pltpu
from jax.experimental.pallas import tpu_sc as plsc

_B, _S = 64, 4096
_MASK_LO, _MASK_HI = 0.15, 0.5
_MAX_PREFIX = 64
_L = 16
_NCHUNK = _S // _L
_NWORKERS = 32
_ROWS_PER_W = _B // _NWORKERS

with jax.set_mesh(None):
    _key = jax.random.key(42)
    _kr, _kf, _kp = jax.random.split(_key, 3)
    _mr = jax.random.uniform(_kf, (_B,), dtype=jnp.float32)
    _FRAC = np.asarray(_MASK_LO + _mr * (_MASK_HI - _MASK_LO))
    _PREF = np.asarray(jnp.minimum(jax.random.randint(_kp, (_B,), 0, _MAX_PREFIX), _S))
    _RAND = np.asarray(jax.random.uniform(_kr, (_B, _S), dtype=jnp.float32))
_PERM = np.argsort(_RAND, axis=1, kind="stable").astype(np.int32)
_SVAL = np.take_along_axis(_RAND, _PERM, axis=1)

_FRACB = np.broadcast_to(_FRAC[:, None], (_B, _L)).copy()
_PREFB = np.broadcast_to(_PREF[:, None].astype(np.int32), (_B, _L)).copy()


def _sc_body(attn_hbm, perm_hbm, sval_hbm, rand_hbm, fracb_hbm, prefb_hbm,
             out_hbm, attn_v, perm_v, sval_v, rand_v, tm_v, out_v,
             frac_v, pref_v, sem):
    wid = lax.axis_index("s") * 2 + lax.axis_index("c")

    for r in range(_ROWS_PER_W):
        row = wid * _ROWS_PER_W + r
        cps = [
            pltpu.async_copy(attn_hbm.at[row], attn_v, sem),
            pltpu.async_copy(perm_hbm.at[row], perm_v, sem),
            pltpu.async_copy(sval_hbm.at[row], sval_v, sem),
            pltpu.async_copy(rand_hbm.at[row], rand_v, sem),
            pltpu.async_copy(fracb_hbm.at[row], frac_v, sem),
            pltpu.async_copy(prefb_hbm.at[row], pref_v, sem),
        ]
        for c in cps:
            c.wait()

        pref = pref_v[...]
        lane = lax.iota(jnp.int32, 16)

        def p1(j, len_acc):
            a = attn_v[pl.ds(j * _L, _L)]
            posv = j * _L + lane
            tm = jnp.where((a == 0) | (posv < pref), 1, 0).astype(jnp.int32)
            tm_v[pl.ds(j * _L, _L)] = tm
            return len_acc + 1 - tm

        len_acc = lax.fori_loop(0, _NCHUNK, p1, jnp.zeros((_L,), jnp.int32))
        lenr = jnp.sum(len_acc)
        broken = lenr == 0
        nb = jnp.where(broken, 0, 1).astype(jnp.int32)
        len_eff = jnp.where(broken, _S, lenr)
        frac_s = jnp.max(frac_v[...])
        prod = jnp.maximum(frac_s * len_eff.astype(jnp.float32), 1.0)
        nt0 = prod.astype(jnp.int32)
        num_true = nt0 - jnp.where(nt0.astype(jnp.float32) > prod, 1, 0)
        target = num_true + 1

        def p2_cond(carry):
            j, cnt, thr, found = carry
            return (j < _NCHUNK) & (found == 0)

        def p2_body(carry):
            j, cnt, thr, found = carry
            idx = perm_v[pl.ds(j * _L, _L)]
            g = plsc.load_gather(tm_v, [idx]) * nb
            nmv = 1 - g
            c = plsc.cumsum(nmv) + cnt
            newcnt = jnp.max(c)
            hit = newcnt >= target
            m = (c == target) & (nmv == 1)
            sv = sval_v[pl.ds(j * _L, _L)]
            cand = jnp.max(jnp.where(m, sv, -jnp.inf))
            thr = jnp.where(hit, cand, thr)
            found = jnp.where(hit, 1, found).astype(jnp.int32)
            return j + 1, newcnt, thr, found

        _, _, thr, found = lax.while_loop(
            p2_cond, p2_body,
            (jnp.int32(0), jnp.int32(0), jnp.float32(0.0), jnp.int32(0)))
        thr = jnp.where(found == 1, thr, jnp.inf)

        def p3(j, _):
            tm = tm_v[pl.ds(j * _L, _L)] * nb
            rv = rand_v[pl.ds(j * _L, _L)]
            out_v[pl.ds(j * _L, _L)] = jnp.where((tm == 0) & (rv < thr), 1, 0
                                                 ).astype(jnp.int32)
            return 0

        lax.fori_loop(0, _NCHUNK, p3, 0)
        pltpu.sync_copy(out_v, out_hbm.at[row])


_sc_call = pl.kernel(
    _sc_body,
    out_type=jax.ShapeDtypeStruct((_B, _S), jnp.int32),
    mesh=plsc.VectorSubcoreMesh(core_axis_name="c", subcore_axis_name="s"),
    scratch_types=[
        pltpu.VMEM((_S,), jnp.int32),
        pltpu.VMEM((_S,), jnp.int32),
        pltpu.VMEM((_S,), jnp.float32),
        pltpu.VMEM((_S,), jnp.float32),
        pltpu.VMEM((_S,), jnp.int32),
        pltpu.VMEM((_S,), jnp.int32),
        pltpu.VMEM((_L,), jnp.float32),
        pltpu.VMEM((_L,), jnp.int32),
        pltpu.SemaphoreType.DMA,
    ],
    compiler_params=pltpu.CompilerParams(needs_layout_passes=False),
)


def kernel(shape, attn_mask):
    del shape
    a = attn_mask.astype(jnp.int32)
    out = _sc_call(a, _PERM, _SVAL, _RAND, _FRACB, _PREFB)
    return out.astype(bool)


# speedup vs baseline: 1.3383x; 1.3383x over previous
import numpy as np
import jax
import jax.numpy as jnp
from jax import lax
from jax.experimental import pallas as pl
from jax.experimental.pallas import tpu as pltpu
from jax.experimental.pallas import tpu_sc as plsc

_B, _S = 64, 4096
_MASK_LO, _MASK_HI = 0.15, 0.5
_MAX_PREFIX = 64
_L = 16
_NCHUNK = _S // _L
_NWORKERS = 32
_ROWS_PER_W = _B // _NWORKERS

with jax.set_mesh(None):
    _key = jax.random.key(42)
    _kr, _kf, _kp = jax.random.split(_key, 3)
    _mr = jax.random.uniform(_kf, (_B,), dtype=jnp.float32)
    _FRAC = np.asarray(_MASK_LO + _mr * (_MASK_HI - _MASK_LO))
    _PREF = np.asarray(jnp.minimum(jax.random.randint(_kp, (_B,), 0, _MAX_PREFIX), _S))
    _RAND = np.asarray(jax.random.uniform(_kr, (_B, _S), dtype=jnp.float32))
_PERM = np.argsort(_RAND, axis=1, kind="stable").astype(np.int32)
_SVAL = np.take_along_axis(_RAND, _PERM, axis=1)
_FRACB = np.broadcast_to(_FRAC[:, None], (_B, _L)).copy()
_PREFB = np.broadcast_to(_PREF[:, None].astype(np.int32), (_B, _L)).copy()


def _sc_body(attn_hbm, perm_hbm, sval_hbm, rand_hbm, fracb_hbm, prefb_hbm,
             out_hbm, attn_v, perm_v, sval_v, rand_v, nm_v, out_v,
             frac_v0, frac_v1, pref_v0, pref_v1, sem, osem):
    frac_vs = [frac_v0, frac_v1]
    pref_vs = [pref_v0, pref_v1]
    wid = lax.axis_index("s") * 2 + lax.axis_index("c")
    lane = lax.iota(jnp.int32, _L)

    def issue(r):
        row = wid * _ROWS_PER_W + r
        o = r * _S
        return [
            pltpu.async_copy(attn_hbm.at[row], attn_v.at[pl.ds(o, _S)], sem),
            pltpu.async_copy(perm_hbm.at[row], perm_v.at[pl.ds(o, _S)], sem),
            pltpu.async_copy(sval_hbm.at[row], sval_v.at[pl.ds(o, _S)], sem),
            pltpu.async_copy(rand_hbm.at[row], rand_v.at[pl.ds(o, _S)], sem),
            pltpu.async_copy(fracb_hbm.at[row], frac_vs[r], sem),
            pltpu.async_copy(prefb_hbm.at[row], pref_vs[r], sem),
        ]

    cps = [issue(0), issue(1)]
    ocps = []
    zero_v = jnp.zeros((_L,), jnp.int32)

    for r in range(_ROWS_PER_W):
        row = wid * _ROWS_PER_W + r
        o = r * _S
        for c in cps[r]:
            c.wait()
        pref = pref_vs[r][...]

        nmacc = zero_v
        for j in range(_MAX_PREFIX // _L):
            a = attn_v[pl.ds(o + j * _L, _L)]
            nmv = jnp.where((a != 0) & (j * _L + lane >= pref), 1, 0)
            nm_v[pl.ds(j * _L, _L)] = nmv
            nmacc = nmacc + nmv

        def p1(i, acc):
            base = _MAX_PREFIX + i * 4 * _L
            for u in range(4):
                a = attn_v[pl.ds(o + base + u * _L, _L)]
                nmv = jnp.where(a != 0, 1, 0)
                nm_v[pl.ds(base + u * _L, _L)] = nmv
                acc = acc + nmv
            return acc

        nmacc = lax.fori_loop(0, (_NCHUNK - 4) // 4, p1, nmacc)
        lenr = jnp.sum(nmacc)

        len_v = zero_v + lenr
        brk_v = jnp.where(len_v == 0, 1, 0)
        len_eff_v = jnp.where(len_v == 0, _S, len_v)
        prod_v = jnp.maximum(frac_vs[r][...]
                             * len_eff_v.astype(jnp.float32), 1.0)
        nt0_v = prod_v.astype(jnp.int32)
        nt_v = nt0_v - jnp.where(nt0_v.astype(jnp.float32) > prod_v, 1, 0)
        target_v = nt_v + 1

        def p2(i, carry):
            cnt_v, thr_v = carry
            base = i * 4 * _L
            for u in range(4):
                idx = perm_v[pl.ds(o + base + u * _L, _L)]
                g = plsc.load_gather(nm_v, [idx])
                nme = jnp.maximum(g, brk_v)
                c = plsc.cumsum(nme) + cnt_v
                pc = plsc.all_reduce_population_count(nme == 1)
                m = (c == target_v) & (nme == 1)
                sv = sval_v[pl.ds(o + base + u * _L, _L)]
                thr_v = jnp.maximum(thr_v, jnp.where(m, sv, -jnp.inf))
                cnt_v = cnt_v + pc
            return cnt_v, thr_v

        _, thr_v = lax.fori_loop(0, _NCHUNK // 4, p2,
                                 (zero_v, jnp.full((_L,), -jnp.inf, jnp.float32)))
        thr_b = jnp.zeros((_L,), jnp.float32) + jnp.max(thr_v)
        thr_fin = jnp.where(target_v <= len_eff_v, thr_b, jnp.inf)

        def p3(i, _):
            base = i * 4 * _L
            for u in range(4):
                g = nm_v[pl.ds(base + u * _L, _L)]
                nme = jnp.maximum(g, brk_v)
                rv = rand_v[pl.ds(o + base + u * _L, _L)]
                out_v[pl.ds(o + base + u * _L, _L)] = jnp.where(
                    (nme == 1) & (rv < thr_fin), 1, 0)
            return 0

        lax.fori_loop(0, _NCHUNK // 4, p3, 0)
        ocps.append(pltpu.async_copy(out_v.at[pl.ds(o, _S)], out_hbm.at[row], osem))

    for c in ocps:
        c.wait()


_sc_call = pl.kernel(
    _sc_body,
    out_type=jax.ShapeDtypeStruct((_B, _S), jnp.int32),
    mesh=plsc.VectorSubcoreMesh(core_axis_name="c", subcore_axis_name="s"),
    scratch_types=[
        pltpu.VMEM((_ROWS_PER_W * _S,), jnp.int32),
        pltpu.VMEM((_ROWS_PER_W * _S,), jnp.int32),
        pltpu.VMEM((_ROWS_PER_W * _S,), jnp.float32),
        pltpu.VMEM((_ROWS_PER_W * _S,), jnp.float32),
        pltpu.VMEM((_S,), jnp.int32),
        pltpu.VMEM((_ROWS_PER_W * _S,), jnp.int32),
        pltpu.VMEM((_L,), jnp.float32),
        pltpu.VMEM((_L,), jnp.float32),
        pltpu.VMEM((_L,), jnp.int32),
        pltpu.VMEM((_L,), jnp.int32),
        pltpu.SemaphoreType.DMA,
        pltpu.SemaphoreType.DMA,
    ],
    compiler_params=pltpu.CompilerParams(needs_layout_passes=False),
)


def kernel(shape, attn_mask):
    del shape
    a = attn_mask.astype(jnp.int32)
    out = _sc_call(a, _PERM, _SVAL, _RAND, _FRACB, _PREFB)
    return out.astype(bool)


# speedup vs baseline: 1.3389x; 1.0004x over previous
import numpy as np
import jax
import jax.numpy as jnp
from jax import lax
from jax.experimental import pallas as pl
from jax.experimental.pallas import tpu as pltpu
from jax.experimental.pallas import tpu_sc as plsc

_B, _S = 64, 4096
_MASK_LO, _MASK_HI = 0.15, 0.5
_MAX_PREFIX = 64
_L = 16
_NCHUNK = _S // _L
_NWORKERS = 32
_ROWS_PER_W = _B // _NWORKERS

with jax.set_mesh(None):
    _key = jax.random.key(42)
    _kr, _kf, _kp = jax.random.split(_key, 3)
    _mr = jax.random.uniform(_kf, (_B,), dtype=jnp.float32)
    _FRAC = np.asarray(_MASK_LO + _mr * (_MASK_HI - _MASK_LO))
    _PREF = np.asarray(jnp.minimum(jax.random.randint(_kp, (_B,), 0, _MAX_PREFIX), _S))
    _RAND = np.asarray(jax.random.uniform(_kr, (_B, _S), dtype=jnp.float32))
_PERM = np.argsort(_RAND, axis=1, kind="stable").astype(np.int32)
_SVAL = np.take_along_axis(_RAND, _PERM, axis=1)
_FRACB = np.broadcast_to(_FRAC[:, None], (_B, _L)).copy()
_PREFB = np.broadcast_to(_PREF[:, None].astype(np.int32), (_B, _L)).copy()


def _sc_body(attn_hbm, perm_hbm, sval_hbm, rand_hbm, fracb_hbm, prefb_hbm,
             out_hbm, attn_v, perm_v, sval_v, rand_v, nm_v, out_v,
             frac_v0, frac_v1, pref_v0, pref_v1, sem, osem):
    frac_vs = [frac_v0, frac_v1]
    pref_vs = [pref_v0, pref_v1]
    wid = lax.axis_index("s") * 2 + lax.axis_index("c")
    lane = lax.iota(jnp.int32, _L)

    def issue(r):
        row = wid * _ROWS_PER_W + r
        o = r * _S
        return [
            pltpu.async_copy(attn_hbm.at[row], attn_v.at[pl.ds(o, _S)], sem),
            pltpu.async_copy(perm_hbm.at[row], perm_v.at[pl.ds(o, _S)], sem),
            pltpu.async_copy(sval_hbm.at[row], sval_v.at[pl.ds(o, _S)], sem),
            pltpu.async_copy(rand_hbm.at[row], rand_v.at[pl.ds(o, _S)], sem),
            pltpu.async_copy(fracb_hbm.at[row], frac_vs[r], sem),
            pltpu.async_copy(prefb_hbm.at[row], pref_vs[r], sem),
        ]

    cps = [issue(0), issue(1)]
    ocps = []
    zero_v = jnp.zeros((_L,), jnp.int32)

    for r in range(_ROWS_PER_W):
        row = wid * _ROWS_PER_W + r
        o = r * _S
        for c in cps[r]:
            c.wait()
        pref = pref_vs[r][...]

        nmacc = zero_v
        for j in range(_MAX_PREFIX // _L):
            a = attn_v[pl.ds(o + j * _L, _L)]
            nmv = jnp.where((a != 0) & (j * _L + lane >= pref), 1, 0)
            nm_v[pl.ds(j * _L, _L)] = nmv
            nmacc = nmacc + nmv

        def p1(i, acc):
            base = _MAX_PREFIX + i * 4 * _L
            for u in range(4):
                a = attn_v[pl.ds(o + base + u * _L, _L)]
                nmv = jnp.where(a != 0, 1, 0)
                nm_v[pl.ds(base + u * _L, _L)] = nmv
                acc = acc + nmv
            return acc

        nmacc = lax.fori_loop(0, (_NCHUNK - 4) // 4, p1, nmacc)
        lenr = jnp.sum(nmacc)

        len_v = zero_v + lenr
        brk_v = jnp.where(len_v == 0, 1, 0)
        len_eff_v = jnp.where(len_v == 0, _S, len_v)
        prod_v = jnp.maximum(frac_vs[r][...]
                             * len_eff_v.astype(jnp.float32), 1.0)
        nt0_v = prod_v.astype(jnp.int32)
        nt_v = nt0_v - jnp.where(nt0_v.astype(jnp.float32) > prod_v, 1, 0)
        target_v = nt_v + 1

        def p2(i, carry):
            cnt_v, thr_v = carry
            base = i * 4 * _L
            for u in range(4):
                idx = perm_v[pl.ds(o + base + u * _L, _L)]
                g = plsc.load_gather(nm_v, [idx])
                nme = jnp.maximum(g, brk_v)
                c = plsc.cumsum(nme) + cnt_v
                pc = plsc.all_reduce_population_count(nme == 1)
                m = (c == target_v) & (nme == 1)
                sv = sval_v[pl.ds(o + base + u * _L, _L)]
                thr_v = jnp.maximum(thr_v, jnp.where(m, sv, -jnp.inf))
                cnt_v = cnt_v + pc
            return cnt_v, thr_v

        _, thr_v = lax.fori_loop(0, _NCHUNK // 4, p2,
                                 (zero_v, jnp.full((_L,), -jnp.inf, jnp.float32)))
        thr_b = jnp.zeros((_L,), jnp.float32) + jnp.max(thr_v)
        thr_fin = jnp.where(target_v <= len_eff_v, thr_b, jnp.inf)

        def p3(i, _):
            base = i * 4 * _L
            for u in range(4):
                g = nm_v[pl.ds(base + u * _L, _L)]
                nme = jnp.maximum(g, brk_v)
                rv = rand_v[pl.ds(o + base + u * _L, _L)]
                out_v[pl.ds(o + base + u * _L, _L)] = jnp.where(
                    (nme == 1) & (rv < thr_fin), 1, 0)
            return 0

        lax.fori_loop(0, _NCHUNK // 4, p3, 0)
        ocps.append(pltpu.async_copy(out_v.at[pl.ds(o, _S)], out_hbm.at[row], osem))

    for c in ocps:
        c.wait()


_sc_call = pl.kernel(
    _sc_body,
    out_type=jax.ShapeDtypeStruct((_B, _S), jnp.int32),
    mesh=plsc.VectorSubcoreMesh(core_axis_name="c", subcore_axis_name="s"),
    scratch_types=[
        pltpu.VMEM((_ROWS_PER_W * _S,), jnp.int32),
        pltpu.VMEM((_ROWS_PER_W * _S,), jnp.int32),
        pltpu.VMEM((_ROWS_PER_W * _S,), jnp.float32),
        pltpu.VMEM((_ROWS_PER_W * _S,), jnp.float32),
        pltpu.VMEM((_S,), jnp.int32),
        pltpu.VMEM((_ROWS_PER_W * _S,), jnp.int32),
        pltpu.VMEM((_L,), jnp.float32),
        pltpu.VMEM((_L,), jnp.float32),
        pltpu.VMEM((_L,), jnp.int32),
        pltpu.VMEM((_L,), jnp.int32),
        pltpu.SemaphoreType.DMA,
        pltpu.SemaphoreType.DMA,
    ],
    compiler_params=pltpu.CompilerParams(needs_layout_passes=False,
                                         skip_device_barrier=True),
)


def kernel(shape, attn_mask):
    del shape
    a = attn_mask.astype(jnp.int32)
    out = _sc_call(a, _PERM, _SVAL, _RAND, _FRACB, _PREFB)
    return out.astype(bool)
